# baseline (speedup 1.0000x reference)
import functools

import jax
import jax.numpy as jnp
from jax import lax
from jax.experimental import pallas as pl
from jax.experimental.pallas import tpu as pltpu
from jax.experimental.pallas import tpu_sc as plsc

D_MODEL = 2048
D_FF = 8192
N_EXP = 8
N_TOK = 4096

TILE_R = 512
N_RTILES = N_TOK // TILE_R

TILE_M = 384
TILE_F = 2048
NF = D_FF // TILE_F
NT_MAX = (2 * N_TOK) // TILE_M + N_EXP
PAD_ROWS = NT_MAX * TILE_M
N_STEPS = NT_MAX * NF
SCHED_COLS = 128

SC_CORES = 2
SC_SUBCORES = 16
NW = SC_CORES * SC_SUBCORES
TOK_PER_W = N_TOK // NW
CH = 32
NCH = TOK_PER_W // CH


def _router_body(x_ref, wr_ref, br_ref, nz_ref,
                 e0_ref, e1_ref, g0_ref, g1_ref, r0_ref, r1_ref, cnt_out_ref,
                 cnt_ref):
    i = pl.program_id(0)

    @pl.when(i == 0)
    def _():
        cnt_ref[...] = jnp.zeros_like(cnt_ref)

    xb = x_ref[...]
    r = jnp.dot(xb, wr_ref[...], preferred_element_type=jnp.float32)
    r = r + br_ref[...]
    logits = r[:, :N_EXP] + nz_ref[...] * jax.nn.softplus(r[:, N_EXP:])

    iota8 = lax.broadcasted_iota(jnp.int32, (TILE_R, N_EXP), 1)
    i1 = jnp.argmax(logits, axis=1).astype(jnp.int32)
    v1 = jnp.max(logits, axis=1, keepdims=True)
    oh1 = iota8 == i1[:, None]
    masked = jnp.where(oh1, -jnp.inf, logits)
    i2 = jnp.argmax(masked, axis=1).astype(jnp.int32)
    v2 = jnp.max(masked, axis=1, keepdims=True)
    oh2 = iota8 == i2[:, None]

    d = jnp.exp(v2 - v1)
    g0 = 1.0 / (1.0 + d)
    g1 = d / (1.0 + d)

    maskf = (oh1 | oh2).astype(jnp.float32)
    ir = lax.broadcasted_iota(jnp.int32, (TILE_R, TILE_R), 0)
    ic = lax.broadcasted_iota(jnp.int32, (TILE_R, TILE_R), 1)
    tril = (ir > ic).astype(jnp.float32)
    cum_exc = jnp.dot(tril, maskf, preferred_element_type=jnp.float32)
    rank = cum_exc + cnt_ref[...]
    r0 = jnp.sum(rank * oh1.astype(jnp.float32), axis=1)
    r1 = jnp.sum(rank * oh2.astype(jnp.float32), axis=1)

    cnt_ref[...] = cnt_ref[...] + jnp.sum(maskf, axis=0, keepdims=True)
    cnt_out_ref[...] = cnt_ref[...]

    e0_ref[...] = i1.reshape(1, 1, TILE_R)
    e1_ref[...] = i2.reshape(1, 1, TILE_R)
    r0_ref[...] = r0.astype(jnp.int32).reshape(1, 1, TILE_R)
    r1_ref[...] = r1.astype(jnp.int32).reshape(1, 1, TILE_R)
    g0_ref[...] = g0.reshape(1, TILE_R, 1)
    g1_ref[...] = g1.reshape(1, TILE_R, 1)


def _router(x2d, wr, br, noise):
    return pl.pallas_call(
        _router_body,
        grid=(N_RTILES,),
        in_specs=[
            pl.BlockSpec((TILE_R, D_MODEL), lambda i: (i, 0)),
            pl.BlockSpec((D_MODEL, 2 * N_EXP), lambda i: (0, 0)),
            pl.BlockSpec((1, 2 * N_EXP), lambda i: (0, 0)),
            pl.BlockSpec((TILE_R, N_EXP), lambda i: (i, 0)),
        ],
        out_specs=[
            pl.BlockSpec((1, 1, TILE_R), lambda i: (i, 0, 0)),
            pl.BlockSpec((1, 1, TILE_R), lambda i: (i, 0, 0)),
            pl.BlockSpec((1, TILE_R, 1), lambda i: (i, 0, 0)),
            pl.BlockSpec((1, TILE_R, 1), lambda i: (i, 0, 0)),
            pl.BlockSpec((1, 1, TILE_R), lambda i: (i, 0, 0)),
            pl.BlockSpec((1, 1, TILE_R), lambda i: (i, 0, 0)),
            pl.BlockSpec((1, N_EXP), lambda i: (0, 0)),
        ],
        out_shape=[
            jax.ShapeDtypeStruct((N_RTILES, 1, TILE_R), jnp.int32),
            jax.ShapeDtypeStruct((N_RTILES, 1, TILE_R), jnp.int32),
            jax.ShapeDtypeStruct((N_RTILES, TILE_R, 1), jnp.float32),
            jax.ShapeDtypeStruct((N_RTILES, TILE_R, 1), jnp.float32),
            jax.ShapeDtypeStruct((N_RTILES, 1, TILE_R), jnp.int32),
            jax.ShapeDtypeStruct((N_RTILES, 1, TILE_R), jnp.int32),
            jax.ShapeDtypeStruct((1, N_EXP), jnp.float32),
        ],
        scratch_shapes=[pltpu.VMEM((1, N_EXP), jnp.float32)],
    )(x2d, wr, br, noise)


def _pos_body(cnt_ref, e0_ref, e1_ref, r0_ref, r1_ref,
              p0_ref, p1_ref, sched_ref):
    cnt = cnt_ref[...].astype(jnp.int32)
    nm = (cnt + (TILE_M - 1)) // TILE_M
    ir = lax.broadcasted_iota(jnp.int32, (N_EXP, N_EXP), 0)
    ic = lax.broadcasted_iota(jnp.int32, (N_EXP, N_EXP), 1)
    triu = (ir <= ic).astype(jnp.float32)
    cum_inc = jnp.dot(nm.astype(jnp.float32), triu,
                      preferred_element_type=jnp.float32).astype(jnp.int32)

    e0 = e0_ref[...]
    e1 = e1_ref[...]
    off0 = jnp.zeros_like(e0)
    off1 = jnp.zeros_like(e1)
    for j in range(N_EXP):
        off_j = (cum_inc[0, j] - nm[0, j]) * TILE_M
        off0 = jnp.where(e0 == j, off_j, off0)
        off1 = jnp.where(e1 == j, off_j, off1)
    p0_ref[...] = off0 + r0_ref[...]
    p1_ref[...] = off1 + r1_ref[...]

    nt = cum_inc[0, N_EXP - 1]
    total = nt * NF
    s = lax.broadcasted_iota(jnp.int32, (1, SCHED_COLS), 1)
    valid = (s < total).astype(jnp.int32)
    sreal = jnp.minimum(s, total - 1)
    idx = jnp.full_like(sreal, -1)
    start_sel = jnp.zeros_like(sreal)
    for j in range(N_EXP):
        cum_ex_j = cum_inc[0, j] - nm[0, j]
        for ff in range(NF):
            start = NF * cum_ex_j + ff * nm[0, j]
            ge = sreal >= start
            idx = idx + ge.astype(jnp.int32)
            start_sel = jnp.where(ge, start, start_sel)
    e_s = idx // NF
    f = idx - e_s * NF
    m = sreal - start_sel
    tile = jnp.zeros_like(sreal)
    for j in range(N_EXP):
        tile = jnp.where(e_s == j, cum_inc[0, j] - nm[0, j] + m, tile)
    zrow = jnp.zeros_like(s)
    sched_ref[...] = jnp.concatenate(
        [e_s, f, tile, valid, zrow, zrow, zrow, zrow], axis=0)


def _pos(cnt, e0, e1, r0, r1):
    return pl.pallas_call(
        _pos_body,
        out_shape=[
            jax.ShapeDtypeStruct((N_RTILES, 1, TILE_R), jnp.int32),
            jax.ShapeDtypeStruct((N_RTILES, 1, TILE_R), jnp.int32),
            jax.ShapeDtypeStruct((8, SCHED_COLS), jnp.int32),
        ],
    )(cnt, e0, e1, r0, r1)


def _scatter_kernel(x2d, p0, p1):
    mesh = plsc.VectorSubcoreMesh(core_axis_name="c", subcore_axis_name="s")

    @functools.partial(
        pl.kernel,
        out_type=jax.ShapeDtypeStruct((PAD_ROWS, D_MODEL), jnp.float32),
        mesh=mesh,
        scratch_types=[
            pltpu.VMEM((CH, D_MODEL), jnp.float32),
            pltpu.VMEM((CH,), jnp.int32),
            pltpu.VMEM((CH,), jnp.int32),
            pltpu.SemaphoreType.DMA,
        ],
    )
    def k(x_hbm, p0_hbm, p1_hbm, xs_hbm, xv, iv0, iv1, sem):
        wid = lax.axis_index("s") * SC_CORES + lax.axis_index("c")

        @pl.loop(0, NCH)
        def _(c):
            base = wid * TOK_PER_W + c * CH
            pltpu.sync_copy(x_hbm.at[pl.ds(base, CH)], xv)
            pltpu.sync_copy(p0_hbm.at[pl.ds(base, CH)], iv0)
            pltpu.sync_copy(p1_hbm.at[pl.ds(base, CH)], iv1)
            pltpu.async_copy(xv, xs_hbm.at[iv0], sem).wait()
            pltpu.async_copy(xv, xs_hbm.at[iv1], sem).wait()

    return k(x2d, p0, p1)


def _gather_kernel(ys, p0, p1):
    mesh = plsc.VectorSubcoreMesh(core_axis_name="c", subcore_axis_name="s")

    @functools.partial(
        pl.kernel,
        out_type=[
            jax.ShapeDtypeStruct((N_TOK, D_MODEL), jnp.float32),
            jax.ShapeDtypeStruct((N_TOK, D_MODEL), jnp.float32),
        ],
        mesh=mesh,
        scratch_types=[
            pltpu.VMEM((CH, D_MODEL), jnp.float32),
            pltpu.VMEM((CH,), jnp.int32),
            pltpu.SemaphoreType.DMA,
        ],
    )
    def k(ys_hbm, p0_hbm, p1_hbm, a_hbm, b_hbm, rv, iv, sem):
        wid = lax.axis_index("s") * SC_CORES + lax.axis_index("c")

        @pl.loop(0, NCH)
        def _(c):
            base = wid * TOK_PER_W + c * CH
            pltpu.sync_copy(p0_hbm.at[pl.ds(base, CH)], iv)
            pltpu.async_copy(ys_hbm.at[iv], rv, sem).wait()
            pltpu.sync_copy(rv, a_hbm.at[pl.ds(base, CH)])
            pltpu.sync_copy(p1_hbm.at[pl.ds(base, CH)], iv)
            pltpu.async_copy(ys_hbm.at[iv], rv, sem).wait()
            pltpu.sync_copy(rv, b_hbm.at[pl.ds(base, CH)])

    return k(ys, p0, p1)


def _ffn1_body(sched_ref, xs_ref, w1_ref, b1_ref, h_ref):
    s = pl.program_id(0)
    valid = sched_ref[3, s] == 1

    @pl.when(valid)
    def _():
        h = jnp.dot(xs_ref[...], w1_ref[0], preferred_element_type=jnp.float32)
        h_ref[...] = jnp.maximum(h + b1_ref[0], 0.0)


def _ffn1(sched, xs, w1, b1):
    grid_spec = pltpu.PrefetchScalarGridSpec(
        num_scalar_prefetch=1,
        grid=(N_STEPS,),
        in_specs=[
            pl.BlockSpec((TILE_M, D_MODEL), lambda s, sr: (sr[2, s], 0)),
            pl.BlockSpec((1, D_MODEL, TILE_F), lambda s, sr: (sr[0, s], 0, sr[1, s])),
            pl.BlockSpec((1, 1, TILE_F), lambda s, sr: (sr[0, s], 0, sr[1, s])),
        ],
        out_specs=pl.BlockSpec((TILE_M, TILE_F), lambda s, sr: (sr[2, s], sr[1, s])),
    )
    return pl.pallas_call(
        _ffn1_body,
        grid_spec=grid_spec,
        out_shape=jax.ShapeDtypeStruct((PAD_ROWS, D_FF), jnp.float32),
    )(sched, xs, w1, b1)


def _ffn2_body(sched_ref, h_ref, w2_ref, b2_ref, part_ref):
    s = pl.program_id(0)
    valid = sched_ref[3, s] == 1

    @pl.when(valid)
    def _():
        y = jnp.dot(h_ref[...], w2_ref[0], preferred_element_type=jnp.float32)
        part_ref[...] = y + b2_ref[0] * (1.0 / NF)


def _ffn2(sched, h, w2, b2):
    grid_spec = pltpu.PrefetchScalarGridSpec(
        num_scalar_prefetch=1,
        grid=(N_STEPS,),
        in_specs=[
            pl.BlockSpec((TILE_M, TILE_F), lambda s, sr: (sr[2, s], sr[1, s])),
            pl.BlockSpec((1, TILE_F, D_MODEL), lambda s, sr: (sr[0, s], sr[1, s], 0)),
            pl.BlockSpec((1, 1, D_MODEL), lambda s, sr: (sr[0, s], 0, 0)),
        ],
        out_specs=pl.BlockSpec((TILE_M, D_MODEL),
                               lambda s, sr: (sr[1, s] * NT_MAX + sr[2, s], 0)),
    )
    return pl.pallas_call(
        _ffn2_body,
        grid_spec=grid_spec,
        out_shape=jax.ShapeDtypeStruct((NF * PAD_ROWS, D_MODEL), jnp.float32),
    )(sched, h, w2, b2)


RB = 128


def _reduce_body(p_ref, o_ref):
    o_ref[...] = jnp.sum(p_ref[...], axis=0)


def _reduce(part3):
    return pl.pallas_call(
        _reduce_body,
        grid=(PAD_ROWS // RB,),
        in_specs=[pl.BlockSpec((NF, RB, D_MODEL), lambda i: (0, i, 0))],
        out_specs=pl.BlockSpec((RB, D_MODEL), lambda i: (i, 0)),
        out_shape=jax.ShapeDtypeStruct((PAD_ROWS, D_MODEL), jnp.float32),
    )(part3)


def _combine_body(a_ref, b_ref, g0_ref, g1_ref, o_ref):
    o_ref[...] = a_ref[...] * g0_ref[0] + b_ref[...] * g1_ref[0]


def _combine(a, b, g0, g1):
    return pl.pallas_call(
        _combine_body,
        grid=(N_RTILES,),
        in_specs=[
            pl.BlockSpec((TILE_R, D_MODEL), lambda i: (i, 0)),
            pl.BlockSpec((TILE_R, D_MODEL), lambda i: (i, 0)),
            pl.BlockSpec((1, TILE_R, 1), lambda i: (i, 0, 0)),
            pl.BlockSpec((1, TILE_R, 1), lambda i: (i, 0, 0)),
        ],
        out_specs=pl.BlockSpec((TILE_R, D_MODEL), lambda i: (i, 0)),
        out_shape=jax.ShapeDtypeStruct((N_TOK, D_MODEL), jnp.float32),
    )(a, b, g0, g1)


@jax.jit
def kernel(x, W_ln, b_ln, W_noise, b_noise, W1, b1, W2, b2):
    x2d = x.reshape(N_TOK, D_MODEL)
    noise = jax.random.normal(jax.random.key(42), x.shape[:-1] + (N_EXP,),
                              dtype=jnp.float32).reshape(N_TOK, N_EXP)
    wr = jnp.concatenate([W_ln, W_noise], axis=1)
    br = jnp.concatenate([b_ln, b_noise]).reshape(1, 2 * N_EXP)

    e0, e1, g0, g1, r0, r1, cnt = _router(x2d, wr, br, noise)
    p0, p1, sched = _pos(cnt, e0, e1, r0, r1)
    p0f = p0.reshape(N_TOK)
    p1f = p1.reshape(N_TOK)

    xs = _scatter_kernel(x2d, p0f, p1f)
    h = _ffn1(sched, xs, W1, b1.reshape(N_EXP, 1, D_FF))
    part = _ffn2(sched, h, W2, b2.reshape(N_EXP, 1, D_MODEL))
    ys = _reduce(part.reshape(NF, PAD_ROWS, D_MODEL))
    a, b = _gather_kernel(ys, p0f, p1f)
    out = _combine(a, b, g0, g1)
    return out.reshape(x.shape)


# speedup vs baseline: 1.0918x; 1.0918x over previous
import functools

import jax
import jax.numpy as jnp
from jax import lax
from jax.experimental import pallas as pl
from jax.experimental.pallas import tpu as pltpu
from jax.experimental.pallas import tpu_sc as plsc

D_MODEL = 2048
D_FF = 8192
N_EXP = 8
N_TOK = 4096

TILE_R = 512
N_RTILES = N_TOK // TILE_R

TILE_M = 384
TILE_F = 512
NF = D_FF // TILE_F
MS = 4
MSUP = MS * TILE_M
SUP_MAX = 3
NT_MAX = (2 * N_TOK) // TILE_M + N_EXP
PAD_ROWS = NT_MAX * TILE_M
N_STEPS = NT_MAX * NF
SCHED_COLS = 512

SC_CORES = 2
SC_SUBCORES = 16
NW = SC_CORES * SC_SUBCORES
TOK_PER_W = N_TOK // NW
CH = 32
NCH = TOK_PER_W // CH


def _router_body(x_ref, wr_ref, br_ref, nz_ref,
                 e0_ref, e1_ref, g0_ref, g1_ref, r0_ref, r1_ref, cnt_out_ref,
                 cnt_ref):
    i = pl.program_id(0)

    @pl.when(i == 0)
    def _():
        cnt_ref[...] = jnp.zeros_like(cnt_ref)

    xb = x_ref[...]
    r = jnp.dot(xb, wr_ref[...], preferred_element_type=jnp.float32)
    r = r + br_ref[...]
    logits = r[:, :N_EXP] + nz_ref[...] * jax.nn.softplus(r[:, N_EXP:])

    iota8 = lax.broadcasted_iota(jnp.int32, (TILE_R, N_EXP), 1)
    i1 = jnp.argmax(logits, axis=1).astype(jnp.int32)
    v1 = jnp.max(logits, axis=1, keepdims=True)
    oh1 = iota8 == i1[:, None]
    masked = jnp.where(oh1, -jnp.inf, logits)
    i2 = jnp.argmax(masked, axis=1).astype(jnp.int32)
    v2 = jnp.max(masked, axis=1, keepdims=True)
    oh2 = iota8 == i2[:, None]

    d = jnp.exp(v2 - v1)
    g0 = 1.0 / (1.0 + d)
    g1 = d / (1.0 + d)

    maskf = (oh1 | oh2).astype(jnp.float32)
    ir = lax.broadcasted_iota(jnp.int32, (TILE_R, TILE_R), 0)
    ic = lax.broadcasted_iota(jnp.int32, (TILE_R, TILE_R), 1)
    tril = (ir > ic).astype(jnp.float32)
    cum_exc = jnp.dot(tril, maskf, preferred_element_type=jnp.float32)
    rank = cum_exc + cnt_ref[...]
    r0 = jnp.sum(rank * oh1.astype(jnp.float32), axis=1)
    r1 = jnp.sum(rank * oh2.astype(jnp.float32), axis=1)

    cnt_ref[...] = cnt_ref[...] + jnp.sum(maskf, axis=0, keepdims=True)
    cnt_out_ref[...] = cnt_ref[...]

    e0_ref[...] = i1.reshape(1, 1, TILE_R)
    e1_ref[...] = i2.reshape(1, 1, TILE_R)
    r0_ref[...] = r0.astype(jnp.int32).reshape(1, 1, TILE_R)
    r1_ref[...] = r1.astype(jnp.int32).reshape(1, 1, TILE_R)
    g0_ref[...] = g0.reshape(1, TILE_R, 1)
    g1_ref[...] = g1.reshape(1, TILE_R, 1)


def _router(x2d, wr, br, noise):
    return pl.pallas_call(
        _router_body,
        grid=(N_RTILES,),
        in_specs=[
            pl.BlockSpec((TILE_R, D_MODEL), lambda i: (i, 0)),
            pl.BlockSpec((D_MODEL, 2 * N_EXP), lambda i: (0, 0)),
            pl.BlockSpec((1, 2 * N_EXP), lambda i: (0, 0)),
            pl.BlockSpec((TILE_R, N_EXP), lambda i: (i, 0)),
        ],
        out_specs=[
            pl.BlockSpec((1, 1, TILE_R), lambda i: (i, 0, 0)),
            pl.BlockSpec((1, 1, TILE_R), lambda i: (i, 0, 0)),
            pl.BlockSpec((1, TILE_R, 1), lambda i: (i, 0, 0)),
            pl.BlockSpec((1, TILE_R, 1), lambda i: (i, 0, 0)),
            pl.BlockSpec((1, 1, TILE_R), lambda i: (i, 0, 0)),
            pl.BlockSpec((1, 1, TILE_R), lambda i: (i, 0, 0)),
            pl.BlockSpec((1, N_EXP), lambda i: (0, 0)),
        ],
        out_shape=[
            jax.ShapeDtypeStruct((N_RTILES, 1, TILE_R), jnp.int32),
            jax.ShapeDtypeStruct((N_RTILES, 1, TILE_R), jnp.int32),
            jax.ShapeDtypeStruct((N_RTILES, TILE_R, 1), jnp.float32),
            jax.ShapeDtypeStruct((N_RTILES, TILE_R, 1), jnp.float32),
            jax.ShapeDtypeStruct((N_RTILES, 1, TILE_R), jnp.int32),
            jax.ShapeDtypeStruct((N_RTILES, 1, TILE_R), jnp.int32),
            jax.ShapeDtypeStruct((1, N_EXP), jnp.float32),
        ],
        scratch_shapes=[pltpu.VMEM((1, N_EXP), jnp.float32)],
    )(x2d, wr, br, noise)


def _pos_body(cnt_ref, e0_ref, e1_ref, r0_ref, r1_ref,
              p0_ref, p1_ref, sched_ref):
    cnt = cnt_ref[...].astype(jnp.int32)
    nm = (cnt + (TILE_M - 1)) // TILE_M
    ir = lax.broadcasted_iota(jnp.int32, (N_EXP, N_EXP), 0)
    ic = lax.broadcasted_iota(jnp.int32, (N_EXP, N_EXP), 1)
    triu = (ir <= ic).astype(jnp.float32)
    cum_inc = jnp.dot(nm.astype(jnp.float32), triu,
                      preferred_element_type=jnp.float32).astype(jnp.int32)

    e0 = e0_ref[...]
    e1 = e1_ref[...]
    off0 = jnp.zeros_like(e0)
    off1 = jnp.zeros_like(e1)
    for j in range(N_EXP):
        off_j = (cum_inc[0, j] - nm[0, j]) * TILE_M
        off0 = jnp.where(e0 == j, off_j, off0)
        off1 = jnp.where(e1 == j, off_j, off1)
    p0_ref[...] = off0 + r0_ref[...]
    p1_ref[...] = off1 + r1_ref[...]

    nt = cum_inc[0, N_EXP - 1]
    total = nt * NF
    s = lax.broadcasted_iota(jnp.int32, (1, SCHED_COLS), 1)
    valid = (s < total).astype(jnp.int32)
    sreal = jnp.minimum(s, total - 1)
    idx = jnp.full_like(sreal, -1)
    start_sel = jnp.zeros_like(sreal)
    nm_sel = jnp.zeros_like(sreal)
    for j in range(N_EXP):
        nm_j = nm[0, j]
        base_j = NF * (cum_inc[0, j] - nm_j)
        for sup in range(SUP_MAX):
            nm_s = jnp.clip(nm_j - MS * sup, 0, MS)
            sup_base = base_j + NF * jnp.minimum(MS * sup, nm_j)
            for ff in range(NF):
                start = sup_base + ff * nm_s
                ge = sreal >= start
                idx = idx + ge.astype(jnp.int32)
                start_sel = jnp.where(ge, start, start_sel)
                nm_sel = jnp.where(ge, nm_s, nm_sel)
    e_s = idx // (SUP_MAX * NF)
    rem = idx - e_s * (SUP_MAX * NF)
    sup_s = rem // NF
    f = rem - sup_s * NF
    m = sreal - start_sel
    cum_ex_sel = jnp.zeros_like(sreal)
    for j in range(N_EXP):
        cum_ex_sel = jnp.where(e_s == j, cum_inc[0, j] - nm[0, j], cum_ex_sel)
    tile = cum_ex_sel + MS * sup_s + m
    tile_last = cum_ex_sel + MS * sup_s + nm_sel - 1
    prev_flushed = jnp.maximum(cum_ex_sel + MS * sup_s - 1, 0)
    xf = jnp.where(f == 0, tile, tile_last)
    oy = jnp.where(f == NF - 1, tile, prev_flushed)
    zrow = jnp.zeros_like(s)
    sched_ref[...] = jnp.concatenate(
        [e_s, f, m, tile, valid, xf, oy, zrow], axis=0)


def _pos(cnt, e0, e1, r0, r1):
    return pl.pallas_call(
        _pos_body,
        out_shape=[
            jax.ShapeDtypeStruct((N_RTILES, 1, TILE_R), jnp.int32),
            jax.ShapeDtypeStruct((N_RTILES, 1, TILE_R), jnp.int32),
            jax.ShapeDtypeStruct((8, SCHED_COLS), jnp.int32),
        ],
    )(cnt, e0, e1, r0, r1)


def _scatter_kernel(x2d, p0, p1):
    mesh = plsc.VectorSubcoreMesh(core_axis_name="c", subcore_axis_name="s")

    @functools.partial(
        pl.kernel,
        out_type=jax.ShapeDtypeStruct((PAD_ROWS, D_MODEL), jnp.float32),
        mesh=mesh,
        scratch_types=[
            pltpu.VMEM((CH, D_MODEL), jnp.float32),
            pltpu.VMEM((CH,), jnp.int32),
            pltpu.VMEM((CH,), jnp.int32),
            pltpu.SemaphoreType.DMA,
        ],
    )
    def k(x_hbm, p0_hbm, p1_hbm, xs_hbm, xv, iv0, iv1, sem):
        wid = lax.axis_index("s") * SC_CORES + lax.axis_index("c")

        @pl.loop(0, NCH)
        def _(c):
            base = wid * TOK_PER_W + c * CH
            pltpu.sync_copy(x_hbm.at[pl.ds(base, CH)], xv)
            pltpu.sync_copy(p0_hbm.at[pl.ds(base, CH)], iv0)
            pltpu.sync_copy(p1_hbm.at[pl.ds(base, CH)], iv1)
            pltpu.async_copy(xv, xs_hbm.at[iv0], sem).wait()
            pltpu.async_copy(xv, xs_hbm.at[iv1], sem).wait()

    return k(x2d, p0, p1)


def _gather_kernel(ys, p0, p1):
    mesh = plsc.VectorSubcoreMesh(core_axis_name="c", subcore_axis_name="s")

    @functools.partial(
        pl.kernel,
        out_type=[
            jax.ShapeDtypeStruct((N_TOK, D_MODEL), jnp.float32),
            jax.ShapeDtypeStruct((N_TOK, D_MODEL), jnp.float32),
        ],
        mesh=mesh,
        scratch_types=[
            pltpu.VMEM((CH, D_MODEL), jnp.float32),
            pltpu.VMEM((CH,), jnp.int32),
            pltpu.SemaphoreType.DMA,
        ],
    )
    def k(ys_hbm, p0_hbm, p1_hbm, a_hbm, b_hbm, rv, iv, sem):
        wid = lax.axis_index("s") * SC_CORES + lax.axis_index("c")

        @pl.loop(0, NCH)
        def _(c):
            base = wid * TOK_PER_W + c * CH
            pltpu.sync_copy(p0_hbm.at[pl.ds(base, CH)], iv)
            pltpu.async_copy(ys_hbm.at[iv], rv, sem).wait()
            pltpu.sync_copy(rv, a_hbm.at[pl.ds(base, CH)])
            pltpu.sync_copy(p1_hbm.at[pl.ds(base, CH)], iv)
            pltpu.async_copy(ys_hbm.at[iv], rv, sem).wait()
            pltpu.sync_copy(rv, b_hbm.at[pl.ds(base, CH)])

    return k(ys, p0, p1)


def _ffn_body(sched_ref, xs_ref, w1_ref, w2_ref, b1_ref, b2_ref, ys_ref,
              xsc_ref, acc_ref):
    s = pl.program_id(0)
    f = sched_ref[1, s]
    m = sched_ref[2, s]
    valid = sched_ref[4, s] == 1

    @pl.when(valid)
    def _():
        base = m * TILE_M

        @pl.when(f == 0)
        def _():
            xsc_ref[pl.ds(base, TILE_M), :] = xs_ref[...]

        xb = xsc_ref[pl.ds(base, TILE_M), :]
        h = jnp.dot(xb, w1_ref[0], preferred_element_type=jnp.float32)
        h = jnp.maximum(h + b1_ref[0], 0.0)
        y = jnp.dot(h, w2_ref[0], preferred_element_type=jnp.float32)

        @pl.when(f == 0)
        def _():
            acc_ref[pl.ds(base, TILE_M), :] = y

        @pl.when(f != 0)
        def _():
            acc_ref[pl.ds(base, TILE_M), :] = acc_ref[pl.ds(base, TILE_M), :] + y

        @pl.when(f == NF - 1)
        def _():
            ys_ref[...] = acc_ref[pl.ds(base, TILE_M), :] + b2_ref[0]


def _ffn(sched, xs, w1, w2, b1, b2):
    grid_spec = pltpu.PrefetchScalarGridSpec(
        num_scalar_prefetch=1,
        grid=(N_STEPS,),
        in_specs=[
            pl.BlockSpec((TILE_M, D_MODEL), lambda s, sr: (sr[5, s], 0)),
            pl.BlockSpec((1, D_MODEL, TILE_F), lambda s, sr: (sr[0, s], 0, sr[1, s])),
            pl.BlockSpec((1, TILE_F, D_MODEL), lambda s, sr: (sr[0, s], sr[1, s], 0)),
            pl.BlockSpec((1, 1, TILE_F), lambda s, sr: (sr[0, s], 0, sr[1, s])),
            pl.BlockSpec((1, 1, D_MODEL), lambda s, sr: (sr[0, s], 0, 0)),
        ],
        out_specs=pl.BlockSpec((TILE_M, D_MODEL), lambda s, sr: (sr[6, s], 0)),
        scratch_shapes=[
            pltpu.VMEM((MSUP, D_MODEL), jnp.float32),
            pltpu.VMEM((MSUP, D_MODEL), jnp.float32),
        ],
    )
    return pl.pallas_call(
        _ffn_body,
        grid_spec=grid_spec,
        out_shape=jax.ShapeDtypeStruct((PAD_ROWS, D_MODEL), jnp.float32),
    )(sched, xs, w1, w2, b1, b2)


def _combine_body(a_ref, b_ref, g0_ref, g1_ref, o_ref):
    o_ref[...] = a_ref[...] * g0_ref[0] + b_ref[...] * g1_ref[0]


def _combine(a, b, g0, g1):
    return pl.pallas_call(
        _combine_body,
        grid=(N_RTILES,),
        in_specs=[
            pl.BlockSpec((TILE_R, D_MODEL), lambda i: (i, 0)),
            pl.BlockSpec((TILE_R, D_MODEL), lambda i: (i, 0)),
            pl.BlockSpec((1, TILE_R, 1), lambda i: (i, 0, 0)),
            pl.BlockSpec((1, TILE_R, 1), lambda i: (i, 0, 0)),
        ],
        out_specs=pl.BlockSpec((TILE_R, D_MODEL), lambda i: (i, 0)),
        out_shape=jax.ShapeDtypeStruct((N_TOK, D_MODEL), jnp.float32),
    )(a, b, g0, g1)


@jax.jit
def kernel(x, W_ln, b_ln, W_noise, b_noise, W1, b1, W2, b2):
    x2d = x.reshape(N_TOK, D_MODEL)
    noise = jax.random.normal(jax.random.key(42), x.shape[:-1] + (N_EXP,),
                              dtype=jnp.float32).reshape(N_TOK, N_EXP)
    wr = jnp.concatenate([W_ln, W_noise], axis=1)
    br = jnp.concatenate([b_ln, b_noise]).reshape(1, 2 * N_EXP)

    e0, e1, g0, g1, r0, r1, cnt = _router(x2d, wr, br, noise)
    p0, p1, sched = _pos(cnt, e0, e1, r0, r1)
    p0f = p0.reshape(N_TOK)
    p1f = p1.reshape(N_TOK)

    xs = _scatter_kernel(x2d, p0f, p1f)
    ys = _ffn(sched, xs, W1, W2,
              b1.reshape(N_EXP, 1, D_FF), b2.reshape(N_EXP, 1, D_MODEL))
    a, b = _gather_kernel(ys, p0f, p1f)
    out = _combine(a, b, g0, g1)
    return out.reshape(x.shape)


# speedup vs baseline: 1.1787x; 1.0795x over previous
import functools

import jax
import jax.numpy as jnp
from jax import lax
from jax.experimental import pallas as pl
from jax.experimental.pallas import tpu as pltpu
from jax.experimental.pallas import tpu_sc as plsc

D_MODEL = 2048
D_FF = 8192
N_EXP = 8
N_TOK = 4096

TILE_R = 512
N_RTILES = N_TOK // TILE_R

TILE_M = 384
TILE_F = 1024
NF = D_FF // TILE_F
MS = 3
MSUP = MS * TILE_M
SUP_MAX = 4
NT_MAX = (2 * N_TOK) // TILE_M + N_EXP
PAD_ROWS = NT_MAX * TILE_M
N_STEPS = NT_MAX * NF
SCHED_COLS = 256

SC_CORES = 2
SC_SUBCORES = 16
NW = SC_CORES * SC_SUBCORES
TOK_PER_W = N_TOK // NW
CH = 32
NCH = TOK_PER_W // CH


def _router_body(x_ref, wr_ref, br_ref, nz_ref,
                 e0_ref, e1_ref, g0_ref, g1_ref, r0_ref, r1_ref, cnt_out_ref,
                 cnt_ref):
    i = pl.program_id(0)

    @pl.when(i == 0)
    def _():
        cnt_ref[...] = jnp.zeros_like(cnt_ref)

    xb = x_ref[...]
    r = jnp.dot(xb, wr_ref[...], preferred_element_type=jnp.float32)
    r = r + br_ref[...]
    logits = r[:, :N_EXP] + nz_ref[...] * jax.nn.softplus(r[:, N_EXP:])

    iota8 = lax.broadcasted_iota(jnp.int32, (TILE_R, N_EXP), 1)
    i1 = jnp.argmax(logits, axis=1).astype(jnp.int32)
    v1 = jnp.max(logits, axis=1, keepdims=True)
    oh1 = iota8 == i1[:, None]
    masked = jnp.where(oh1, -jnp.inf, logits)
    i2 = jnp.argmax(masked, axis=1).astype(jnp.int32)
    v2 = jnp.max(masked, axis=1, keepdims=True)
    oh2 = iota8 == i2[:, None]

    d = jnp.exp(v2 - v1)
    g0 = 1.0 / (1.0 + d)
    g1 = d / (1.0 + d)

    maskf = (oh1 | oh2).astype(jnp.float32)
    ir = lax.broadcasted_iota(jnp.int32, (TILE_R, TILE_R), 0)
    ic = lax.broadcasted_iota(jnp.int32, (TILE_R, TILE_R), 1)
    tril = (ir > ic).astype(jnp.float32)
    cum_exc = jnp.dot(tril, maskf, preferred_element_type=jnp.float32)
    rank = cum_exc + cnt_ref[...]
    r0 = jnp.sum(rank * oh1.astype(jnp.float32), axis=1)
    r1 = jnp.sum(rank * oh2.astype(jnp.float32), axis=1)

    cnt_ref[...] = cnt_ref[...] + jnp.sum(maskf, axis=0, keepdims=True)
    cnt_out_ref[...] = cnt_ref[...]

    e0_ref[...] = i1.reshape(1, 1, TILE_R)
    e1_ref[...] = i2.reshape(1, 1, TILE_R)
    r0_ref[...] = r0.astype(jnp.int32).reshape(1, 1, TILE_R)
    r1_ref[...] = r1.astype(jnp.int32).reshape(1, 1, TILE_R)
    g0_ref[...] = g0.reshape(1, TILE_R, 1)
    g1_ref[...] = g1.reshape(1, TILE_R, 1)


def _router(x2d, wr, br, noise):
    return pl.pallas_call(
        _router_body,
        grid=(N_RTILES,),
        in_specs=[
            pl.BlockSpec((TILE_R, D_MODEL), lambda i: (i, 0)),
            pl.BlockSpec((D_MODEL, 2 * N_EXP), lambda i: (0, 0)),
            pl.BlockSpec((1, 2 * N_EXP), lambda i: (0, 0)),
            pl.BlockSpec((TILE_R, N_EXP), lambda i: (i, 0)),
        ],
        out_specs=[
            pl.BlockSpec((1, 1, TILE_R), lambda i: (i, 0, 0)),
            pl.BlockSpec((1, 1, TILE_R), lambda i: (i, 0, 0)),
            pl.BlockSpec((1, TILE_R, 1), lambda i: (i, 0, 0)),
            pl.BlockSpec((1, TILE_R, 1), lambda i: (i, 0, 0)),
            pl.BlockSpec((1, 1, TILE_R), lambda i: (i, 0, 0)),
            pl.BlockSpec((1, 1, TILE_R), lambda i: (i, 0, 0)),
            pl.BlockSpec((1, N_EXP), lambda i: (0, 0)),
        ],
        out_shape=[
            jax.ShapeDtypeStruct((N_RTILES, 1, TILE_R), jnp.int32),
            jax.ShapeDtypeStruct((N_RTILES, 1, TILE_R), jnp.int32),
            jax.ShapeDtypeStruct((N_RTILES, TILE_R, 1), jnp.float32),
            jax.ShapeDtypeStruct((N_RTILES, TILE_R, 1), jnp.float32),
            jax.ShapeDtypeStruct((N_RTILES, 1, TILE_R), jnp.int32),
            jax.ShapeDtypeStruct((N_RTILES, 1, TILE_R), jnp.int32),
            jax.ShapeDtypeStruct((1, N_EXP), jnp.float32),
        ],
        scratch_shapes=[pltpu.VMEM((1, N_EXP), jnp.float32)],
    )(x2d, wr, br, noise)


def _pos_body(cnt_ref, e0_ref, e1_ref, r0_ref, r1_ref,
              p0_ref, p1_ref, sched_ref):
    cnt = cnt_ref[...].astype(jnp.int32)
    nm = (cnt + (TILE_M - 1)) // TILE_M
    ir = lax.broadcasted_iota(jnp.int32, (N_EXP, N_EXP), 0)
    ic = lax.broadcasted_iota(jnp.int32, (N_EXP, N_EXP), 1)
    triu = (ir <= ic).astype(jnp.float32)
    cum_inc = jnp.dot(nm.astype(jnp.float32), triu,
                      preferred_element_type=jnp.float32).astype(jnp.int32)

    e0 = e0_ref[...]
    e1 = e1_ref[...]
    off0 = jnp.zeros_like(e0)
    off1 = jnp.zeros_like(e1)
    for j in range(N_EXP):
        off_j = (cum_inc[0, j] - nm[0, j]) * TILE_M
        off0 = jnp.where(e0 == j, off_j, off0)
        off1 = jnp.where(e1 == j, off_j, off1)
    p0_ref[...] = off0 + r0_ref[...]
    p1_ref[...] = off1 + r1_ref[...]

    nt = cum_inc[0, N_EXP - 1]
    total = nt * NF
    s = lax.broadcasted_iota(jnp.int32, (1, SCHED_COLS), 1)
    valid = (s < total).astype(jnp.int32)
    sreal = jnp.minimum(s, total - 1)
    idx = jnp.full_like(sreal, -1)
    start_sel = jnp.zeros_like(sreal)
    nm_sel = jnp.zeros_like(sreal)
    for j in range(N_EXP):
        nm_j = nm[0, j]
        base_j = NF * (cum_inc[0, j] - nm_j)
        for sup in range(SUP_MAX):
            nm_s = jnp.clip(nm_j - MS * sup, 0, MS)
            sup_base = base_j + NF * jnp.minimum(MS * sup, nm_j)
            for ff in range(NF):
                start = sup_base + ff * nm_s
                ge = sreal >= start
                idx = idx + ge.astype(jnp.int32)
                start_sel = jnp.where(ge, start, start_sel)
                nm_sel = jnp.where(ge, nm_s, nm_sel)
    e_s = idx // (SUP_MAX * NF)
    rem = idx - e_s * (SUP_MAX * NF)
    sup_s = rem // NF
    f = rem - sup_s * NF
    m = sreal - start_sel
    cum_ex_sel = jnp.zeros_like(sreal)
    for j in range(N_EXP):
        cum_ex_sel = jnp.where(e_s == j, cum_inc[0, j] - nm[0, j], cum_ex_sel)
    tile = cum_ex_sel + MS * sup_s + m
    tile_last = cum_ex_sel + MS * sup_s + nm_sel - 1
    prev_flushed = jnp.maximum(cum_ex_sel + MS * sup_s - 1, 0)
    xf = jnp.where(f == 0, tile, tile_last)
    oy = jnp.where(f == NF - 1, tile, prev_flushed)
    zrow = jnp.zeros_like(s)
    sched_ref[...] = jnp.concatenate(
        [e_s, f, m, tile, valid, xf, oy, zrow], axis=0)


def _pos(cnt, e0, e1, r0, r1):
    return pl.pallas_call(
        _pos_body,
        out_shape=[
            jax.ShapeDtypeStruct((N_RTILES, 1, TILE_R), jnp.int32),
            jax.ShapeDtypeStruct((N_RTILES, 1, TILE_R), jnp.int32),
            jax.ShapeDtypeStruct((8, SCHED_COLS), jnp.int32),
        ],
    )(cnt, e0, e1, r0, r1)


def _scatter_kernel(x2d, p0, p1):
    mesh = plsc.VectorSubcoreMesh(core_axis_name="c", subcore_axis_name="s")

    @functools.partial(
        pl.kernel,
        out_type=jax.ShapeDtypeStruct((PAD_ROWS, D_MODEL), jnp.float32),
        mesh=mesh,
        scratch_types=[
            pltpu.VMEM((CH, D_MODEL), jnp.float32),
            pltpu.VMEM((CH,), jnp.int32),
            pltpu.VMEM((CH,), jnp.int32),
            pltpu.SemaphoreType.DMA,
        ],
    )
    def k(x_hbm, p0_hbm, p1_hbm, xs_hbm, xv, iv0, iv1, sem):
        wid = lax.axis_index("s") * SC_CORES + lax.axis_index("c")

        @pl.loop(0, NCH)
        def _(c):
            base = wid * TOK_PER_W + c * CH
            pltpu.sync_copy(x_hbm.at[pl.ds(base, CH)], xv)
            pltpu.sync_copy(p0_hbm.at[pl.ds(base, CH)], iv0)
            pltpu.sync_copy(p1_hbm.at[pl.ds(base, CH)], iv1)
            pltpu.async_copy(xv, xs_hbm.at[iv0], sem).wait()
            pltpu.async_copy(xv, xs_hbm.at[iv1], sem).wait()

    return k(x2d, p0, p1)


def _gather_kernel(ys, p0, p1):
    mesh = plsc.VectorSubcoreMesh(core_axis_name="c", subcore_axis_name="s")

    @functools.partial(
        pl.kernel,
        out_type=[
            jax.ShapeDtypeStruct((N_TOK, D_MODEL), jnp.float32),
            jax.ShapeDtypeStruct((N_TOK, D_MODEL), jnp.float32),
        ],
        mesh=mesh,
        scratch_types=[
            pltpu.VMEM((CH, D_MODEL), jnp.float32),
            pltpu.VMEM((CH,), jnp.int32),
            pltpu.SemaphoreType.DMA,
        ],
    )
    def k(ys_hbm, p0_hbm, p1_hbm, a_hbm, b_hbm, rv, iv, sem):
        wid = lax.axis_index("s") * SC_CORES + lax.axis_index("c")

        @pl.loop(0, NCH)
        def _(c):
            base = wid * TOK_PER_W + c * CH
            pltpu.sync_copy(p0_hbm.at[pl.ds(base, CH)], iv)
            pltpu.async_copy(ys_hbm.at[iv], rv, sem).wait()
            pltpu.sync_copy(rv, a_hbm.at[pl.ds(base, CH)])
            pltpu.sync_copy(p1_hbm.at[pl.ds(base, CH)], iv)
            pltpu.async_copy(ys_hbm.at[iv], rv, sem).wait()
            pltpu.sync_copy(rv, b_hbm.at[pl.ds(base, CH)])

    return k(ys, p0, p1)


def _ffn_body(sched_ref, xs_ref, w1_ref, w2_ref, b1_ref, b2_ref, ys_ref,
              acc_ref):
    s = pl.program_id(0)
    f = sched_ref[1, s]
    m = sched_ref[2, s]
    valid = sched_ref[4, s] == 1

    @pl.when(valid)
    def _():
        base = m * TILE_M
        xb = xs_ref[...]
        h = jnp.dot(xb, w1_ref[0], preferred_element_type=jnp.float32)
        h = jnp.maximum(h + b1_ref[0], 0.0)
        y = jnp.dot(h, w2_ref[0], preferred_element_type=jnp.float32)

        @pl.when(f == 0)
        def _():
            acc_ref[pl.ds(base, TILE_M), :] = y

        @pl.when(f != 0)
        def _():
            acc_ref[pl.ds(base, TILE_M), :] = acc_ref[pl.ds(base, TILE_M), :] + y

        @pl.when(f == NF - 1)
        def _():
            ys_ref[...] = acc_ref[pl.ds(base, TILE_M), :] + b2_ref[0]


def _ffn(sched, xs, w1, w2, b1, b2):
    grid_spec = pltpu.PrefetchScalarGridSpec(
        num_scalar_prefetch=1,
        grid=(N_STEPS,),
        in_specs=[
            pl.BlockSpec((TILE_M, D_MODEL), lambda s, sr: (sr[3, s], 0)),
            pl.BlockSpec((1, D_MODEL, TILE_F), lambda s, sr: (sr[0, s], 0, sr[1, s])),
            pl.BlockSpec((1, TILE_F, D_MODEL), lambda s, sr: (sr[0, s], sr[1, s], 0)),
            pl.BlockSpec((1, 1, TILE_F), lambda s, sr: (sr[0, s], 0, sr[1, s])),
            pl.BlockSpec((1, 1, D_MODEL), lambda s, sr: (sr[0, s], 0, 0)),
        ],
        out_specs=pl.BlockSpec((TILE_M, D_MODEL), lambda s, sr: (sr[6, s], 0)),
        scratch_shapes=[
            pltpu.VMEM((MSUP, D_MODEL), jnp.float32),
        ],
    )
    return pl.pallas_call(
        _ffn_body,
        grid_spec=grid_spec,
        out_shape=jax.ShapeDtypeStruct((PAD_ROWS, D_MODEL), jnp.float32),
    )(sched, xs, w1, w2, b1, b2)


def _combine_body(a_ref, b_ref, g0_ref, g1_ref, o_ref):
    o_ref[...] = a_ref[...] * g0_ref[0] + b_ref[...] * g1_ref[0]


def _combine(a, b, g0, g1):
    return pl.pallas_call(
        _combine_body,
        grid=(N_RTILES,),
        in_specs=[
            pl.BlockSpec((TILE_R, D_MODEL), lambda i: (i, 0)),
            pl.BlockSpec((TILE_R, D_MODEL), lambda i: (i, 0)),
            pl.BlockSpec((1, TILE_R, 1), lambda i: (i, 0, 0)),
            pl.BlockSpec((1, TILE_R, 1), lambda i: (i, 0, 0)),
        ],
        out_specs=pl.BlockSpec((TILE_R, D_MODEL), lambda i: (i, 0)),
        out_shape=jax.ShapeDtypeStruct((N_TOK, D_MODEL), jnp.float32),
    )(a, b, g0, g1)


@jax.jit
def kernel(x, W_ln, b_ln, W_noise, b_noise, W1, b1, W2, b2):
    x2d = x.reshape(N_TOK, D_MODEL)
    noise = jax.random.normal(jax.random.key(42), x.shape[:-1] + (N_EXP,),
                              dtype=jnp.float32).reshape(N_TOK, N_EXP)
    wr = jnp.concatenate([W_ln, W_noise], axis=1)
    br = jnp.concatenate([b_ln, b_noise]).reshape(1, 2 * N_EXP)

    e0, e1, g0, g1, r0, r1, cnt = _router(x2d, wr, br, noise)
    p0, p1, sched = _pos(cnt, e0, e1, r0, r1)
    p0f = p0.reshape(N_TOK)
    p1f = p1.reshape(N_TOK)

    xs = _scatter_kernel(x2d, p0f, p1f)
    ys = _ffn(sched, xs, W1, W2,
              b1.reshape(N_EXP, 1, D_FF), b2.reshape(N_EXP, 1, D_MODEL))
    a, b = _gather_kernel(ys, p0f, p1f)
    out = _combine(a, b, g0, g1)
    return out.reshape(x.shape)


# speedup vs baseline: 1.3804x; 1.1711x over previous
import functools

import jax
import jax.numpy as jnp
from jax import lax
from jax.experimental import pallas as pl
from jax.experimental.pallas import tpu as pltpu
from jax.experimental.pallas import tpu_sc as plsc

D_MODEL = 2048
D_FF = 8192
N_EXP = 8
N_TOK = 4096

TILE_R = 512
N_RTILES = N_TOK // TILE_R

TILE_M = 368
TILE_F = 1024
NF = D_FF // TILE_F
MS = 3
MSUP = MS * TILE_M
SUP_MAX = 4
NT_MAX = (2 * N_TOK) // TILE_M + N_EXP
PAD_ROWS = NT_MAX * TILE_M
N_STEPS = NT_MAX * NF + 1
SCHED_COLS = 256

SC_CORES = 2
SC_SUBCORES = 16
NW = SC_CORES * SC_SUBCORES
TOK_PER_W = N_TOK // NW
CH = 32
NCH = TOK_PER_W // CH


def _router_body(x_ref, wr_ref, br_ref, nz_ref,
                 e0_ref, e1_ref, g0_ref, g1_ref, r0_ref, r1_ref, cnt_out_ref,
                 cnt_ref):
    i = pl.program_id(0)

    @pl.when(i == 0)
    def _():
        cnt_ref[...] = jnp.zeros_like(cnt_ref)

    xb = x_ref[...]
    r = jnp.dot(xb, wr_ref[...], preferred_element_type=jnp.float32)
    r = r + br_ref[...]
    logits = r[:, :N_EXP] + nz_ref[...] * jax.nn.softplus(r[:, N_EXP:])

    iota8 = lax.broadcasted_iota(jnp.int32, (TILE_R, N_EXP), 1)
    i1 = jnp.argmax(logits, axis=1).astype(jnp.int32)
    v1 = jnp.max(logits, axis=1, keepdims=True)
    oh1 = iota8 == i1[:, None]
    masked = jnp.where(oh1, -jnp.inf, logits)
    i2 = jnp.argmax(masked, axis=1).astype(jnp.int32)
    v2 = jnp.max(masked, axis=1, keepdims=True)
    oh2 = iota8 == i2[:, None]

    d = jnp.exp(v2 - v1)
    g0 = 1.0 / (1.0 + d)
    g1 = d / (1.0 + d)

    maskf = (oh1 | oh2).astype(jnp.float32)
    ir = lax.broadcasted_iota(jnp.int32, (TILE_R, TILE_R), 0)
    ic = lax.broadcasted_iota(jnp.int32, (TILE_R, TILE_R), 1)
    tril = (ir > ic).astype(jnp.float32)
    cum_exc = jnp.dot(tril, maskf, preferred_element_type=jnp.float32)
    rank = cum_exc + cnt_ref[...]
    r0 = jnp.sum(rank * oh1.astype(jnp.float32), axis=1)
    r1 = jnp.sum(rank * oh2.astype(jnp.float32), axis=1)

    cnt_ref[...] = cnt_ref[...] + jnp.sum(maskf, axis=0, keepdims=True)
    cnt_out_ref[...] = cnt_ref[...]

    e0_ref[...] = i1.reshape(1, 1, TILE_R)
    e1_ref[...] = i2.reshape(1, 1, TILE_R)
    r0_ref[...] = r0.astype(jnp.int32).reshape(1, 1, TILE_R)
    r1_ref[...] = r1.astype(jnp.int32).reshape(1, 1, TILE_R)
    g0_ref[...] = g0.reshape(1, TILE_R, 1)
    g1_ref[...] = g1.reshape(1, TILE_R, 1)


def _router(x2d, wr, br, noise):
    return pl.pallas_call(
        _router_body,
        grid=(N_RTILES,),
        in_specs=[
            pl.BlockSpec((TILE_R, D_MODEL), lambda i: (i, 0)),
            pl.BlockSpec((D_MODEL, 2 * N_EXP), lambda i: (0, 0)),
            pl.BlockSpec((1, 2 * N_EXP), lambda i: (0, 0)),
            pl.BlockSpec((TILE_R, N_EXP), lambda i: (i, 0)),
        ],
        out_specs=[
            pl.BlockSpec((1, 1, TILE_R), lambda i: (i, 0, 0)),
            pl.BlockSpec((1, 1, TILE_R), lambda i: (i, 0, 0)),
            pl.BlockSpec((1, TILE_R, 1), lambda i: (i, 0, 0)),
            pl.BlockSpec((1, TILE_R, 1), lambda i: (i, 0, 0)),
            pl.BlockSpec((1, 1, TILE_R), lambda i: (i, 0, 0)),
            pl.BlockSpec((1, 1, TILE_R), lambda i: (i, 0, 0)),
            pl.BlockSpec((1, N_EXP), lambda i: (0, 0)),
        ],
        out_shape=[
            jax.ShapeDtypeStruct((N_RTILES, 1, TILE_R), jnp.int32),
            jax.ShapeDtypeStruct((N_RTILES, 1, TILE_R), jnp.int32),
            jax.ShapeDtypeStruct((N_RTILES, TILE_R, 1), jnp.float32),
            jax.ShapeDtypeStruct((N_RTILES, TILE_R, 1), jnp.float32),
            jax.ShapeDtypeStruct((N_RTILES, 1, TILE_R), jnp.int32),
            jax.ShapeDtypeStruct((N_RTILES, 1, TILE_R), jnp.int32),
            jax.ShapeDtypeStruct((1, N_EXP), jnp.float32),
        ],
        scratch_shapes=[pltpu.VMEM((1, N_EXP), jnp.float32)],
    )(x2d, wr, br, noise)


def _pos_body(cnt_ref, e0_ref, e1_ref, r0_ref, r1_ref,
              p0_ref, p1_ref, sched_ref):
    cnt = cnt_ref[...].astype(jnp.int32)
    nm = (cnt + (TILE_M - 1)) // TILE_M
    ir = lax.broadcasted_iota(jnp.int32, (N_EXP, N_EXP), 0)
    ic = lax.broadcasted_iota(jnp.int32, (N_EXP, N_EXP), 1)
    triu = (ir <= ic).astype(jnp.float32)
    cum_inc = jnp.dot(nm.astype(jnp.float32), triu,
                      preferred_element_type=jnp.float32).astype(jnp.int32)

    e0 = e0_ref[...]
    e1 = e1_ref[...]
    off0 = jnp.zeros_like(e0)
    off1 = jnp.zeros_like(e1)
    for j in range(N_EXP):
        off_j = (cum_inc[0, j] - nm[0, j]) * TILE_M
        off0 = jnp.where(e0 == j, off_j, off0)
        off1 = jnp.where(e1 == j, off_j, off1)
    p0_ref[...] = off0 + r0_ref[...]
    p1_ref[...] = off1 + r1_ref[...]

    nt = cum_inc[0, N_EXP - 1]
    total = nt * NF
    s = lax.broadcasted_iota(jnp.int32, (2, SCHED_COLS), 1)
    s = s - lax.broadcasted_iota(jnp.int32, (2, SCHED_COLS), 0)
    valid = ((s >= 0) & (s < total)).astype(jnp.int32)
    sreal = jnp.clip(s, 0, total - 1)
    idx = jnp.full_like(sreal, -1)
    start_sel = jnp.zeros_like(sreal)
    for j in range(N_EXP):
        nm_j = nm[0, j]
        base_j = NF * (cum_inc[0, j] - nm_j)
        for sup in range(SUP_MAX):
            nm_s = jnp.clip(nm_j - MS * sup, 0, MS)
            sup_base = base_j + NF * jnp.minimum(MS * sup, nm_j)
            for ff in range(NF):
                start = sup_base + ff * nm_s
                ge = sreal >= start
                idx = idx + ge.astype(jnp.int32)
                start_sel = jnp.where(ge, start, start_sel)
    e_s = idx // (SUP_MAX * NF)
    rem = idx - e_s * (SUP_MAX * NF)
    sup_s = rem // NF
    f = rem - sup_s * NF
    m = sreal - start_sel
    cum_ex_sel = jnp.zeros_like(sreal)
    for j in range(N_EXP):
        cum_ex_sel = jnp.where(e_s == j, cum_inc[0, j] - nm[0, j], cum_ex_sel)
    tile = cum_ex_sel + MS * sup_s + m
    prev_flushed = jnp.maximum(cum_ex_sel + MS * sup_s - 1, 0)
    oy = jnp.where(f == NF - 1, tile, prev_flushed)
    sched_ref[...] = jnp.concatenate(
        [e_s[:1], f[:1], m[:1], tile[:1], valid[:1],
         e_s[1:], f[1:], m[1:], oy[1:], valid[1:],
         jnp.zeros((6, SCHED_COLS), jnp.int32)], axis=0)


def _pos(cnt, e0, e1, r0, r1):
    return pl.pallas_call(
        _pos_body,
        out_shape=[
            jax.ShapeDtypeStruct((N_RTILES, 1, TILE_R), jnp.int32),
            jax.ShapeDtypeStruct((N_RTILES, 1, TILE_R), jnp.int32),
            jax.ShapeDtypeStruct((16, SCHED_COLS), jnp.int32),
        ],
    )(cnt, e0, e1, r0, r1)


def _scatter_kernel(x2d, p0, p1):
    mesh = plsc.VectorSubcoreMesh(core_axis_name="c", subcore_axis_name="s")

    @functools.partial(
        pl.kernel,
        out_type=jax.ShapeDtypeStruct((PAD_ROWS, D_MODEL), jnp.float32),
        mesh=mesh,
        scratch_types=[
            pltpu.VMEM((CH, D_MODEL), jnp.float32),
            pltpu.VMEM((CH,), jnp.int32),
            pltpu.VMEM((CH,), jnp.int32),
            pltpu.SemaphoreType.DMA,
        ],
    )
    def k(x_hbm, p0_hbm, p1_hbm, xs_hbm, xv, iv0, iv1, sem):
        wid = lax.axis_index("s") * SC_CORES + lax.axis_index("c")

        @pl.loop(0, NCH)
        def _(c):
            base = wid * TOK_PER_W + c * CH
            pltpu.sync_copy(x_hbm.at[pl.ds(base, CH)], xv)
            pltpu.sync_copy(p0_hbm.at[pl.ds(base, CH)], iv0)
            pltpu.sync_copy(p1_hbm.at[pl.ds(base, CH)], iv1)
            pltpu.async_copy(xv, xs_hbm.at[iv0], sem).wait()
            pltpu.async_copy(xv, xs_hbm.at[iv1], sem).wait()

    return k(x2d, p0, p1)


def _gather_kernel(ys, p0, p1):
    mesh = plsc.VectorSubcoreMesh(core_axis_name="c", subcore_axis_name="s")

    @functools.partial(
        pl.kernel,
        out_type=[
            jax.ShapeDtypeStruct((N_TOK, D_MODEL), jnp.float32),
            jax.ShapeDtypeStruct((N_TOK, D_MODEL), jnp.float32),
        ],
        mesh=mesh,
        scratch_types=[
            pltpu.VMEM((CH, D_MODEL), jnp.float32),
            pltpu.VMEM((CH,), jnp.int32),
            pltpu.SemaphoreType.DMA,
        ],
    )
    def k(ys_hbm, p0_hbm, p1_hbm, a_hbm, b_hbm, rv, iv, sem):
        wid = lax.axis_index("s") * SC_CORES + lax.axis_index("c")

        @pl.loop(0, NCH)
        def _(c):
            base = wid * TOK_PER_W + c * CH
            pltpu.sync_copy(p0_hbm.at[pl.ds(base, CH)], iv)
            pltpu.async_copy(ys_hbm.at[iv], rv, sem).wait()
            pltpu.sync_copy(rv, a_hbm.at[pl.ds(base, CH)])
            pltpu.sync_copy(p1_hbm.at[pl.ds(base, CH)], iv)
            pltpu.async_copy(ys_hbm.at[iv], rv, sem).wait()
            pltpu.sync_copy(rv, b_hbm.at[pl.ds(base, CH)])

    return k(ys, p0, p1)


def _ffn_body(sched_ref, xs_ref, w1_ref, w2_ref, b1_ref, b2_ref, ys_ref,
              acc_ref, h_ref):
    s = pl.program_id(0)
    valid = sched_ref[4, s] == 1
    valid2 = sched_ref[9, s] == 1
    par = lax.rem(s, 2)

    @pl.when(valid)
    def _():
        h = jnp.dot(xs_ref[...], w1_ref[0], preferred_element_type=jnp.float32)
        h_ref[pl.ds(par * TILE_M, TILE_M), :] = jnp.maximum(h + b1_ref[0], 0.0)

    @pl.when(valid2)
    def _():
        f2 = sched_ref[6, s]
        base2 = sched_ref[7, s] * TILE_M
        hp = h_ref[pl.ds((1 - par) * TILE_M, TILE_M), :]
        y = jnp.dot(hp, w2_ref[0], preferred_element_type=jnp.float32)

        @pl.when(f2 == 0)
        def _():
            acc_ref[pl.ds(base2, TILE_M), :] = y

        @pl.when(f2 != 0)
        def _():
            acc_ref[pl.ds(base2, TILE_M), :] = (
                acc_ref[pl.ds(base2, TILE_M), :] + y)

        @pl.when(f2 == NF - 1)
        def _():
            ys_ref[...] = acc_ref[pl.ds(base2, TILE_M), :] + b2_ref[0]


def _ffn(sched, xs, w1, w2, b1, b2):
    grid_spec = pltpu.PrefetchScalarGridSpec(
        num_scalar_prefetch=1,
        grid=(N_STEPS,),
        in_specs=[
            pl.BlockSpec((TILE_M, D_MODEL), lambda s, sr: (sr[3, s], 0)),
            pl.BlockSpec((1, D_MODEL, TILE_F), lambda s, sr: (sr[0, s], 0, sr[1, s])),
            pl.BlockSpec((1, TILE_F, D_MODEL), lambda s, sr: (sr[5, s], sr[6, s], 0)),
            pl.BlockSpec((1, 1, TILE_F), lambda s, sr: (sr[0, s], 0, sr[1, s])),
            pl.BlockSpec((1, 1, D_MODEL), lambda s, sr: (sr[5, s], 0, 0)),
        ],
        out_specs=pl.BlockSpec((TILE_M, D_MODEL), lambda s, sr: (sr[8, s], 0)),
        scratch_shapes=[
            pltpu.VMEM((MSUP, D_MODEL), jnp.float32),
            pltpu.VMEM((2 * TILE_M, TILE_F), jnp.float32),
        ],
    )
    return pl.pallas_call(
        _ffn_body,
        grid_spec=grid_spec,
        out_shape=jax.ShapeDtypeStruct((PAD_ROWS, D_MODEL), jnp.float32),
    )(sched, xs, w1, w2, b1, b2)


def _combine_body(a_ref, b_ref, g0_ref, g1_ref, o_ref):
    o_ref[...] = a_ref[...] * g0_ref[0] + b_ref[...] * g1_ref[0]


def _combine(a, b, g0, g1):
    return pl.pallas_call(
        _combine_body,
        grid=(N_RTILES,),
        in_specs=[
            pl.BlockSpec((TILE_R, D_MODEL), lambda i: (i, 0)),
            pl.BlockSpec((TILE_R, D_MODEL), lambda i: (i, 0)),
            pl.BlockSpec((1, TILE_R, 1), lambda i: (i, 0, 0)),
            pl.BlockSpec((1, TILE_R, 1), lambda i: (i, 0, 0)),
        ],
        out_specs=pl.BlockSpec((TILE_R, D_MODEL), lambda i: (i, 0)),
        out_shape=jax.ShapeDtypeStruct((N_TOK, D_MODEL), jnp.float32),
    )(a, b, g0, g1)


@jax.jit
def kernel(x, W_ln, b_ln, W_noise, b_noise, W1, b1, W2, b2):
    x2d = x.reshape(N_TOK, D_MODEL)
    noise = jax.random.normal(jax.random.key(42), x.shape[:-1] + (N_EXP,),
                              dtype=jnp.float32).reshape(N_TOK, N_EXP)
    wr = jnp.concatenate([W_ln, W_noise], axis=1)
    br = jnp.concatenate([b_ln, b_noise]).reshape(1, 2 * N_EXP)

    e0, e1, g0, g1, r0, r1, cnt = _router(x2d, wr, br, noise)
    p0, p1, sched = _pos(cnt, e0, e1, r0, r1)
    p0f = p0.reshape(N_TOK)
    p1f = p1.reshape(N_TOK)

    xs = _scatter_kernel(x2d, p0f, p1f)
    ys = _ffn(sched, xs, W1, W2,
              b1.reshape(N_EXP, 1, D_FF), b2.reshape(N_EXP, 1, D_MODEL))
    a, b = _gather_kernel(ys, p0f, p1f)
    out = _combine(a, b, g0, g1)
    return out.reshape(x.shape)


# speedup vs baseline: 1.3908x; 1.0076x over previous
import functools

import jax
import jax.numpy as jnp
from jax import lax
from jax.experimental import pallas as pl
from jax.experimental.pallas import tpu as pltpu
from jax.experimental.pallas import tpu_sc as plsc

D_MODEL = 2048
D_FF = 8192
N_EXP = 8
N_TOK = 4096

TILE_R = 512
N_RTILES = N_TOK // TILE_R

TILE_M = 360
TILE_F = 1024
NF = D_FF // TILE_F
MS = 3
MSUP = MS * TILE_M
SUP_MAX = 4
NT_MAX = (2 * N_TOK) // TILE_M + N_EXP
PAD_ROWS = NT_MAX * TILE_M
N_STEPS = NT_MAX * NF + 1
SCHED_COLS = 256

SC_CORES = 2
SC_SUBCORES = 16
NW = SC_CORES * SC_SUBCORES
TOK_PER_W = N_TOK // NW
CH = 32
NCH = TOK_PER_W // CH


def _router_body(x_ref, wr_ref, br_ref, nz_ref,
                 e0_ref, e1_ref, g0_ref, g1_ref, r0_ref, r1_ref, cnt_out_ref,
                 cnt_ref):
    i = pl.program_id(0)

    @pl.when(i == 0)
    def _():
        cnt_ref[...] = jnp.zeros_like(cnt_ref)

    xb = x_ref[...]
    r = jnp.dot(xb, wr_ref[...], preferred_element_type=jnp.float32)
    r = r + br_ref[...]
    logits = r[:, :N_EXP] + nz_ref[...] * jax.nn.softplus(r[:, N_EXP:])

    iota8 = lax.broadcasted_iota(jnp.int32, (TILE_R, N_EXP), 1)
    i1 = jnp.argmax(logits, axis=1).astype(jnp.int32)
    v1 = jnp.max(logits, axis=1, keepdims=True)
    oh1 = iota8 == i1[:, None]
    masked = jnp.where(oh1, -jnp.inf, logits)
    i2 = jnp.argmax(masked, axis=1).astype(jnp.int32)
    v2 = jnp.max(masked, axis=1, keepdims=True)
    oh2 = iota8 == i2[:, None]

    d = jnp.exp(v2 - v1)
    g0 = 1.0 / (1.0 + d)
    g1 = d / (1.0 + d)

    maskf = (oh1 | oh2).astype(jnp.float32)
    ir = lax.broadcasted_iota(jnp.int32, (TILE_R, TILE_R), 0)
    ic = lax.broadcasted_iota(jnp.int32, (TILE_R, TILE_R), 1)
    tril = (ir > ic).astype(jnp.float32)
    cum_exc = jnp.dot(tril, maskf, preferred_element_type=jnp.float32)
    rank = cum_exc + cnt_ref[...]
    r0 = jnp.sum(rank * oh1.astype(jnp.float32), axis=1)
    r1 = jnp.sum(rank * oh2.astype(jnp.float32), axis=1)

    cnt_ref[...] = cnt_ref[...] + jnp.sum(maskf, axis=0, keepdims=True)
    cnt_out_ref[...] = cnt_ref[...]

    e0_ref[...] = i1.reshape(1, 1, TILE_R)
    e1_ref[...] = i2.reshape(1, 1, TILE_R)
    r0_ref[...] = r0.astype(jnp.int32).reshape(1, 1, TILE_R)
    r1_ref[...] = r1.astype(jnp.int32).reshape(1, 1, TILE_R)
    g0_ref[...] = g0.reshape(1, TILE_R, 1)
    g1_ref[...] = g1.reshape(1, TILE_R, 1)


def _router(x2d, wr, br, noise):
    return pl.pallas_call(
        _router_body,
        grid=(N_RTILES,),
        in_specs=[
            pl.BlockSpec((TILE_R, D_MODEL), lambda i: (i, 0)),
            pl.BlockSpec((D_MODEL, 2 * N_EXP), lambda i: (0, 0)),
            pl.BlockSpec((1, 2 * N_EXP), lambda i: (0, 0)),
            pl.BlockSpec((TILE_R, N_EXP), lambda i: (i, 0)),
        ],
        out_specs=[
            pl.BlockSpec((1, 1, TILE_R), lambda i: (i, 0, 0)),
            pl.BlockSpec((1, 1, TILE_R), lambda i: (i, 0, 0)),
            pl.BlockSpec((1, TILE_R, 1), lambda i: (i, 0, 0)),
            pl.BlockSpec((1, TILE_R, 1), lambda i: (i, 0, 0)),
            pl.BlockSpec((1, 1, TILE_R), lambda i: (i, 0, 0)),
            pl.BlockSpec((1, 1, TILE_R), lambda i: (i, 0, 0)),
            pl.BlockSpec((1, N_EXP), lambda i: (0, 0)),
        ],
        out_shape=[
            jax.ShapeDtypeStruct((N_RTILES, 1, TILE_R), jnp.int32),
            jax.ShapeDtypeStruct((N_RTILES, 1, TILE_R), jnp.int32),
            jax.ShapeDtypeStruct((N_RTILES, TILE_R, 1), jnp.float32),
            jax.ShapeDtypeStruct((N_RTILES, TILE_R, 1), jnp.float32),
            jax.ShapeDtypeStruct((N_RTILES, 1, TILE_R), jnp.int32),
            jax.ShapeDtypeStruct((N_RTILES, 1, TILE_R), jnp.int32),
            jax.ShapeDtypeStruct((1, N_EXP), jnp.float32),
        ],
        scratch_shapes=[pltpu.VMEM((1, N_EXP), jnp.float32)],
    )(x2d, wr, br, noise)


def _pos_body(cnt_ref, e0_ref, e1_ref, r0_ref, r1_ref,
              p0_ref, p1_ref, sched_ref):
    cnt = cnt_ref[...].astype(jnp.int32)
    nm = (cnt + (TILE_M - 1)) // TILE_M
    ir = lax.broadcasted_iota(jnp.int32, (N_EXP, N_EXP), 0)
    ic = lax.broadcasted_iota(jnp.int32, (N_EXP, N_EXP), 1)
    triu = (ir <= ic).astype(jnp.float32)
    cum_inc = jnp.dot(nm.astype(jnp.float32), triu,
                      preferred_element_type=jnp.float32).astype(jnp.int32)

    e0 = e0_ref[...]
    e1 = e1_ref[...]
    off0 = jnp.zeros_like(e0)
    off1 = jnp.zeros_like(e1)
    for j in range(N_EXP):
        off_j = (cum_inc[0, j] - nm[0, j]) * TILE_M
        off0 = jnp.where(e0 == j, off_j, off0)
        off1 = jnp.where(e1 == j, off_j, off1)
    p0_ref[...] = off0 + r0_ref[...]
    p1_ref[...] = off1 + r1_ref[...]

    nt = cum_inc[0, N_EXP - 1]
    total = nt * NF
    s = lax.broadcasted_iota(jnp.int32, (2, SCHED_COLS), 1)
    s = s - lax.broadcasted_iota(jnp.int32, (2, SCHED_COLS), 0)
    valid = ((s >= 0) & (s < total)).astype(jnp.int32)
    sreal = jnp.clip(s, 0, total - 1)
    idx = jnp.full_like(sreal, -1)
    start_sel = jnp.zeros_like(sreal)
    for j in range(N_EXP):
        nm_j = nm[0, j]
        base_j = NF * (cum_inc[0, j] - nm_j)
        for sup in range(SUP_MAX):
            nm_s = jnp.clip(nm_j - MS * sup, 0, MS)
            sup_base = base_j + NF * jnp.minimum(MS * sup, nm_j)
            for ff in range(NF):
                start = sup_base + ff * nm_s
                ge = sreal >= start
                idx = idx + ge.astype(jnp.int32)
                start_sel = jnp.where(ge, start, start_sel)
    e_s = idx // (SUP_MAX * NF)
    rem = idx - e_s * (SUP_MAX * NF)
    sup_s = rem // NF
    f = rem - sup_s * NF
    m = sreal - start_sel
    cum_ex_sel = jnp.zeros_like(sreal)
    for j in range(N_EXP):
        cum_ex_sel = jnp.where(e_s == j, cum_inc[0, j] - nm[0, j], cum_ex_sel)
    tile = cum_ex_sel + MS * sup_s + m
    prev_flushed = jnp.maximum(cum_ex_sel + MS * sup_s - 1, 0)
    oy = jnp.where(f == NF - 1, tile, prev_flushed)
    sched_ref[...] = jnp.concatenate(
        [e_s[:1], f[:1], m[:1], tile[:1], valid[:1],
         e_s[1:], f[1:], m[1:], oy[1:], valid[1:],
         jnp.zeros((6, SCHED_COLS), jnp.int32)], axis=0)


def _pos(cnt, e0, e1, r0, r1):
    return pl.pallas_call(
        _pos_body,
        out_shape=[
            jax.ShapeDtypeStruct((N_RTILES, 1, TILE_R), jnp.int32),
            jax.ShapeDtypeStruct((N_RTILES, 1, TILE_R), jnp.int32),
            jax.ShapeDtypeStruct((16, SCHED_COLS), jnp.int32),
        ],
    )(cnt, e0, e1, r0, r1)


def _scatter_kernel(x2d, p0, p1):
    mesh = plsc.VectorSubcoreMesh(core_axis_name="c", subcore_axis_name="s")

    @functools.partial(
        pl.kernel,
        out_type=jax.ShapeDtypeStruct((PAD_ROWS, D_MODEL), jnp.float32),
        mesh=mesh,
        scratch_types=[
            pltpu.VMEM((CH, D_MODEL), jnp.float32),
            pltpu.VMEM((CH,), jnp.int32),
            pltpu.VMEM((CH,), jnp.int32),
            pltpu.SemaphoreType.DMA,
        ],
    )
    def k(x_hbm, p0_hbm, p1_hbm, xs_hbm, xv, iv0, iv1, sem):
        wid = lax.axis_index("s") * SC_CORES + lax.axis_index("c")

        @pl.loop(0, NCH)
        def _(c):
            base = wid * TOK_PER_W + c * CH
            pltpu.sync_copy(x_hbm.at[pl.ds(base, CH)], xv)
            pltpu.sync_copy(p0_hbm.at[pl.ds(base, CH)], iv0)
            pltpu.sync_copy(p1_hbm.at[pl.ds(base, CH)], iv1)
            pltpu.async_copy(xv, xs_hbm.at[iv0], sem).wait()
            pltpu.async_copy(xv, xs_hbm.at[iv1], sem).wait()

    return k(x2d, p0, p1)


def _gather_kernel(ys, p0, p1):
    mesh = plsc.VectorSubcoreMesh(core_axis_name="c", subcore_axis_name="s")

    @functools.partial(
        pl.kernel,
        out_type=[
            jax.ShapeDtypeStruct((N_TOK, D_MODEL), jnp.float32),
            jax.ShapeDtypeStruct((N_TOK, D_MODEL), jnp.float32),
        ],
        mesh=mesh,
        scratch_types=[
            pltpu.VMEM((CH, D_MODEL), jnp.float32),
            pltpu.VMEM((CH,), jnp.int32),
            pltpu.SemaphoreType.DMA,
        ],
    )
    def k(ys_hbm, p0_hbm, p1_hbm, a_hbm, b_hbm, rv, iv, sem):
        wid = lax.axis_index("s") * SC_CORES + lax.axis_index("c")

        @pl.loop(0, NCH)
        def _(c):
            base = wid * TOK_PER_W + c * CH
            pltpu.sync_copy(p0_hbm.at[pl.ds(base, CH)], iv)
            pltpu.async_copy(ys_hbm.at[iv], rv, sem).wait()
            pltpu.sync_copy(rv, a_hbm.at[pl.ds(base, CH)])
            pltpu.sync_copy(p1_hbm.at[pl.ds(base, CH)], iv)
            pltpu.async_copy(ys_hbm.at[iv], rv, sem).wait()
            pltpu.sync_copy(rv, b_hbm.at[pl.ds(base, CH)])

    return k(ys, p0, p1)


def _ffn_body(sched_ref, xs_ref, w1_ref, w2_ref, b1_ref, b2_ref, ys_ref,
              acc_ref, h_ref):
    s = pl.program_id(0)
    valid = sched_ref[4, s] == 1
    valid2 = sched_ref[9, s] == 1
    par = lax.rem(s, 2)

    @pl.when(valid)
    def _():
        h = jnp.dot(xs_ref[...], w1_ref[0], preferred_element_type=jnp.float32)
        h_ref[pl.ds(par * TILE_M, TILE_M), :] = jnp.maximum(h + b1_ref[0], 0.0)

    @pl.when(valid2)
    def _():
        f2 = sched_ref[6, s]
        base2 = sched_ref[7, s] * TILE_M
        hp = h_ref[pl.ds((1 - par) * TILE_M, TILE_M), :]
        y = jnp.dot(hp, w2_ref[0], preferred_element_type=jnp.float32)

        @pl.when(f2 == 0)
        def _():
            acc_ref[pl.ds(base2, TILE_M), :] = y

        @pl.when(f2 != 0)
        def _():
            acc_ref[pl.ds(base2, TILE_M), :] = (
                acc_ref[pl.ds(base2, TILE_M), :] + y)

        @pl.when(f2 == NF - 1)
        def _():
            ys_ref[...] = acc_ref[pl.ds(base2, TILE_M), :] + b2_ref[0]


def _ffn(sched, xs, w1, w2, b1, b2):
    grid_spec = pltpu.PrefetchScalarGridSpec(
        num_scalar_prefetch=1,
        grid=(N_STEPS,),
        in_specs=[
            pl.BlockSpec((TILE_M, D_MODEL), lambda s, sr: (sr[3, s], 0)),
            pl.BlockSpec((1, D_MODEL, TILE_F), lambda s, sr: (sr[0, s], 0, sr[1, s])),
            pl.BlockSpec((1, TILE_F, D_MODEL), lambda s, sr: (sr[5, s], sr[6, s], 0)),
            pl.BlockSpec((1, 1, TILE_F), lambda s, sr: (sr[0, s], 0, sr[1, s])),
            pl.BlockSpec((1, 1, D_MODEL), lambda s, sr: (sr[5, s], 0, 0)),
        ],
        out_specs=pl.BlockSpec((TILE_M, D_MODEL), lambda s, sr: (sr[8, s], 0)),
        scratch_shapes=[
            pltpu.VMEM((MSUP, D_MODEL), jnp.float32),
            pltpu.VMEM((2 * TILE_M, TILE_F), jnp.float32),
        ],
    )
    return pl.pallas_call(
        _ffn_body,
        grid_spec=grid_spec,
        out_shape=jax.ShapeDtypeStruct((PAD_ROWS, D_MODEL), jnp.float32),
    )(sched, xs, w1, w2, b1, b2)


def _combine_body(a_ref, b_ref, g0_ref, g1_ref, o_ref):
    o_ref[...] = a_ref[...] * g0_ref[0] + b_ref[...] * g1_ref[0]


def _combine(a, b, g0, g1):
    return pl.pallas_call(
        _combine_body,
        grid=(N_RTILES,),
        in_specs=[
            pl.BlockSpec((TILE_R, D_MODEL), lambda i: (i, 0)),
            pl.BlockSpec((TILE_R, D_MODEL), lambda i: (i, 0)),
            pl.BlockSpec((1, TILE_R, 1), lambda i: (i, 0, 0)),
            pl.BlockSpec((1, TILE_R, 1), lambda i: (i, 0, 0)),
        ],
        out_specs=pl.BlockSpec((TILE_R, D_MODEL), lambda i: (i, 0)),
        out_shape=jax.ShapeDtypeStruct((N_TOK, D_MODEL), jnp.float32),
    )(a, b, g0, g1)


@jax.jit
def kernel(x, W_ln, b_ln, W_noise, b_noise, W1, b1, W2, b2):
    x2d = x.reshape(N_TOK, D_MODEL)
    noise = jax.random.normal(jax.random.key(42), x.shape[:-1] + (N_EXP,),
                              dtype=jnp.float32).reshape(N_TOK, N_EXP)
    wr = jnp.concatenate([W_ln, W_noise], axis=1)
    br = jnp.concatenate([b_ln, b_noise]).reshape(1, 2 * N_EXP)

    e0, e1, g0, g1, r0, r1, cnt = _router(x2d, wr, br, noise)
    p0, p1, sched = _pos(cnt, e0, e1, r0, r1)
    p0f = p0.reshape(N_TOK)
    p1f = p1.reshape(N_TOK)

    xs = _scatter_kernel(x2d, p0f, p1f)
    ys = _ffn(sched, xs, W1, W2,
              b1.reshape(N_EXP, 1, D_FF), b2.reshape(N_EXP, 1, D_MODEL))
    a, b = _gather_kernel(ys, p0f, p1f)
    out = _combine(a, b, g0, g1)
    return out.reshape(x.shape)
